# Initial kernel scaffold; baseline (speedup 1.0000x reference)
import jax
import jax.numpy as jnp
from jax.experimental import pallas as pl


def kernel(x, edge_index, batch, W1, b1, W2, b2, W3, b3, W4, b4, fcW1, fcb1, fcW2, fcb2):
    raise NotImplementedError("write your pallas kernel here")


# speedup vs baseline: 7.9787x; 7.9787x over previous
import functools

import jax
import jax.numpy as jnp
from jax import lax
from jax.experimental import pallas as pl
from jax.experimental.pallas import tpu as pltpu
from jax.experimental.pallas import tpu_sc as plsc

_N = 10000
_E = 320000
_G = 16
_CW = 128
_NTILES = 16
_NCORES = 2
_NW = _NCORES * _NTILES
_EPT = _E // _NW
_K = 80
_NCHUNK = _EPT // _K
_RPT = 640
_NPAD = _RPT * _NTILES

_PREC = lax.Precision.HIGHEST


def _sc_mesh():
  return plsc.VectorSubcoreMesh(core_axis_name="c", subcore_axis_name="s")


@functools.partial(
    pl.kernel,
    out_type=jax.ShapeDtypeStruct((_NCORES, _NPAD, _CW), jnp.float32),
    mesh=_sc_mesh(),
    scratch_types=[
        pltpu.VMEM((_K,), jnp.int32),
        pltpu.VMEM((_K, _CW), jnp.float32),
        pltpu.VMEM((_K, _CW), jnp.float32),
        pltpu.VMEM_SHARED((_NPAD, _CW), jnp.float32),
    ],
)
def _sc_degree(dst_hbm, out_hbm, di_v, ones_v, zb_v, acc_sh):
  c = lax.axis_index("c")
  s = lax.axis_index("s")
  tid = c * _NTILES + s
  base = tid * _EPT
  zeros16 = jnp.zeros((16,), jnp.float32)
  ones16 = jnp.ones((16,), jnp.float32)

  def fill(i, carry):
    r = i // (_CW // 16)
    col = (i % (_CW // 16)) * 16
    ones_v[r, pl.ds(col, 16)] = ones16
    zb_v[r, pl.ds(col, 16)] = zeros16
    return carry
  lax.fori_loop(0, _K * _CW // 16, fill, 0)

  for k in range(_RPT // _K):
    pltpu.sync_copy(zb_v, acc_sh.at[pl.ds(s * _RPT + k * _K, _K)])
  plsc.subcore_barrier()

  def chunk(j, carry):
    pltpu.sync_copy(dst_hbm.at[pl.ds(base + j * _K, _K)], di_v)
    pltpu.sync_copy(ones_v, acc_sh.at[di_v], add=True)
    return carry
  lax.fori_loop(0, _NCHUNK, chunk, 0)

  plsc.subcore_barrier()
  pltpu.sync_copy(acc_sh.at[pl.ds(s * _RPT, _RPT)],
                  out_hbm.at[c, pl.ds(s * _RPT, _RPT)])


def _make_sc_spmm(ncb):
  @functools.partial(
      pl.kernel,
      out_type=jax.ShapeDtypeStruct((_NCORES, ncb, _NPAD, _CW), jnp.float32),
      mesh=_sc_mesh(),
      scratch_types=[
          pltpu.VMEM((_K,), jnp.int32),
          pltpu.VMEM((_K,), jnp.int32),
          pltpu.VMEM((_K, _CW), jnp.float32),
          pltpu.VMEM((_K, _CW), jnp.float32),
          pltpu.VMEM_SHARED((_NPAD, _CW), jnp.float32),
          pltpu.SemaphoreType.DMA,
      ],
  )
  def spmm(src_hbm, dst_hbm, u_hbm, out_hbm, si_v, di_v, rows_v, zb_v,
           acc_sh, sem):
    c = lax.axis_index("c")
    s = lax.axis_index("s")
    tid = c * _NTILES + s
    base = tid * _EPT
    zeros16 = jnp.zeros((16,), jnp.float32)

    def zero_zb(i, carry):
      r = i // (_CW // 16)
      col = (i % (_CW // 16)) * 16
      zb_v[r, pl.ds(col, 16)] = zeros16
      return carry
    lax.fori_loop(0, _K * _CW // 16, zero_zb, 0)

    for p in range(ncb):
      for k in range(_RPT // _K):
        pltpu.sync_copy(zb_v, acc_sh.at[pl.ds(s * _RPT + k * _K, _K)])
      plsc.subcore_barrier()

      def chunk(j, carry):
        off = base + j * _K
        pltpu.sync_copy(src_hbm.at[pl.ds(off, _K)], si_v)
        pltpu.sync_copy(dst_hbm.at[pl.ds(off, _K)], di_v)
        pltpu.async_copy(u_hbm.at[p].at[si_v], rows_v, sem).wait()
        pltpu.sync_copy(rows_v, acc_sh.at[di_v], add=True)
        return carry
      lax.fori_loop(0, _NCHUNK, chunk, 0)

      plsc.subcore_barrier()
      pltpu.sync_copy(acc_sh.at[pl.ds(s * _RPT, _RPT)],
                      out_hbm.at[c, p, pl.ds(s * _RPT, _RPT)])
      if p + 1 < ncb:
        plsc.subcore_barrier()

  return spmm


_SC_SPMM = {ncb: _make_sc_spmm(ncb) for ncb in (1, 2, 4)}


_BN = 1000
_GRID = _N // _BN


def _prep_body(x_ref, dega_ref, degb_ref, dis_ref, u1_ref):
  deg = dega_ref[...] + degb_ref[...] + 1.0
  dis = lax.rsqrt(deg)
  dis_ref[...] = jnp.broadcast_to(dis, (_BN, 128))
  u1_ref[0] = x_ref[...] * dis


def _tc_prep(x, dega, degb):
  return pl.pallas_call(
      _prep_body,
      grid=(_GRID,),
      in_specs=[
          pl.BlockSpec((_BN, 128), lambda i: (i, 0)),
          pl.BlockSpec((_BN, 1), lambda i: (i, 0)),
          pl.BlockSpec((_BN, 1), lambda i: (i, 0)),
      ],
      out_specs=[
          pl.BlockSpec((_BN, 128), lambda i: (i, 0)),
          pl.BlockSpec((1, _BN, 128), lambda i: (0, i, 0)),
      ],
      out_shape=[
          jax.ShapeDtypeStruct((_N, 128), jnp.float32),
          jax.ShapeDtypeStruct((1, _N, 128), jnp.float32),
      ],
  )(x, dega, degb)


def _make_layer_body(ncb_in, ncb_out):
  def body(s_ref, u_ref, dis_ref, w_ref, b_ref, uo_ref):
    cols = [s_ref[0, p] + s_ref[1, p] + u_ref[p] for p in range(ncb_in)]
    hcat = cols[0] if ncb_in == 1 else jnp.concatenate(cols, axis=1)
    dis = dis_ref[:, 0:1]
    agg = hcat * dis
    h = jnp.dot(agg, w_ref[...], preferred_element_type=jnp.float32,
                precision=_PREC) + b_ref[...]
    un = jnp.maximum(h, 0.0) * dis
    for p in range(ncb_out):
      uo_ref[p] = un[:, p * _CW:(p + 1) * _CW]
  return body


def _tc_layer(s_p, u, dis, W, b, ncb_in, ncb_out):
  d_in = ncb_in * _CW
  d_out = ncb_out * _CW
  return pl.pallas_call(
      _make_layer_body(ncb_in, ncb_out),
      grid=(_GRID,),
      in_specs=[
          pl.BlockSpec((2, ncb_in, _BN, _CW), lambda i: (0, 0, i, 0)),
          pl.BlockSpec((ncb_in, _BN, _CW), lambda i: (0, i, 0)),
          pl.BlockSpec((_BN, 128), lambda i: (i, 0)),
          pl.BlockSpec((d_in, d_out), lambda i: (0, 0)),
          pl.BlockSpec((1, d_out), lambda i: (0, 0)),
      ],
      out_specs=pl.BlockSpec((ncb_out, _BN, _CW), lambda i: (0, i, 0)),
      out_shape=jax.ShapeDtypeStruct((ncb_out, _N, _CW), jnp.float32),
  )(s_p, u, dis, W, b.reshape(1, d_out))


def _head_prep_body(fcw1_ref, fcw2_ref, fcb1_ref, fcb2_ref, w_ref, c0_ref):
  w_ref[...] = jnp.dot(fcw1_ref[...], fcw2_ref[...],
                       preferred_element_type=jnp.float32, precision=_PREC)
  c0_ref[...] = jnp.dot(fcb1_ref[...], fcw2_ref[...],
                        preferred_element_type=jnp.float32,
                        precision=_PREC) + fcb2_ref[...]


def _tc_head_prep(fcW1, fcb1, fcW2, fcb2):
  return pl.pallas_call(
      _head_prep_body,
      out_shape=[
          jax.ShapeDtypeStruct((1024, 1), jnp.float32),
          jax.ShapeDtypeStruct((1, 1), jnp.float32),
      ],
  )(fcW1, fcW2, fcb1.reshape(1, 512), fcb2.reshape(1, 1))


def _layer4_body(s_ref, u_ref, dis_ref, w_ref, b_ref, wh_ref, z_ref):
  cols = [s_ref[0, p] + s_ref[1, p] + u_ref[p] for p in range(4)]
  hcat = jnp.concatenate(cols, axis=1)
  dis = dis_ref[:, 0:1]
  agg = hcat * dis
  h = jnp.dot(agg, w_ref[...], preferred_element_type=jnp.float32,
              precision=_PREC) + b_ref[...]
  h = jnp.maximum(h, 0.0)
  z_ref[...] = jnp.dot(h, wh_ref[...], preferred_element_type=jnp.float32,
                       precision=_PREC)


def _tc_layer4(s_p, u, dis, W4, b4, w_head):
  return pl.pallas_call(
      _layer4_body,
      grid=(_GRID,),
      in_specs=[
          pl.BlockSpec((2, 4, _BN, _CW), lambda i: (0, 0, i, 0)),
          pl.BlockSpec((4, _BN, _CW), lambda i: (0, i, 0)),
          pl.BlockSpec((_BN, 128), lambda i: (i, 0)),
          pl.BlockSpec((512, 1024), lambda i: (0, 0)),
          pl.BlockSpec((1, 1024), lambda i: (0, 0)),
          pl.BlockSpec((1024, 1), lambda i: (0, 0)),
      ],
      out_specs=pl.BlockSpec((_BN, 1), lambda i: (i, 0)),
      out_shape=jax.ShapeDtypeStruct((_N, 1), jnp.float32),
  )(s_p, u, dis, W4, b4.reshape(1, 1024), w_head)


def _pool_body(z_ref, batch_ref, c0_ref, out_ref, scr):
  i = pl.program_id(0)

  @pl.when(i == 0)
  def _():
    scr[...] = jnp.zeros_like(scr)

  gid = lax.broadcasted_iota(jnp.int32, (_G, _BN), 0)
  bmat = (batch_ref[:, 0][None, :] == gid).astype(jnp.float32)
  zsum = jnp.dot(bmat, z_ref[...], preferred_element_type=jnp.float32,
                 precision=_PREC)
  cnt = jnp.sum(bmat, axis=1, keepdims=True)
  scr[:, 0:1] += zsum
  scr[:, 1:2] += cnt

  @pl.when(i == _GRID - 1)
  def _():
    out_ref[...] = (scr[:, 0:1] / jnp.maximum(scr[:, 1:2], 1.0)
                    + c0_ref[...])


def _tc_pool(z, batch2d, c0):
  return pl.pallas_call(
      _pool_body,
      grid=(_GRID,),
      in_specs=[
          pl.BlockSpec((_BN, 1), lambda i: (i, 0)),
          pl.BlockSpec((_BN, 1), lambda i: (i, 0)),
          pl.BlockSpec((1, 1), lambda i: (0, 0)),
      ],
      out_specs=pl.BlockSpec((_G, 1), lambda i: (0, 0)),
      out_shape=jax.ShapeDtypeStruct((_G, 1), jnp.float32),
      scratch_shapes=[pltpu.VMEM((_G, 2), jnp.float32)],
  )(z, batch2d, c0)


@jax.jit
def kernel(x, edge_index, batch, W1, b1, W2, b2, W3, b3, W4, b4,
           fcW1, fcb1, fcW2, fcb2):
  src = edge_index[0]
  dst = edge_index[1]

  degp = _sc_degree(dst)
  dega = degp[0, :_N, 0].reshape(_N, 1)
  degb = degp[1, :_N, 0].reshape(_N, 1)
  dis, u = _tc_prep(x, dega, degb)

  w_head, c0 = _tc_head_prep(fcW1, fcb1, fcW2, fcb2)

  dims = [(1, 1, W1, b1), (1, 2, W2, b2), (2, 4, W3, b3)]
  for ncb_in, ncb_out, W, b in dims:
    s_p = _SC_SPMM[ncb_in](src, dst, u)
    u = _tc_layer(s_p, u, dis, W, b, ncb_in, ncb_out)

  s_p = _SC_SPMM[4](src, dst, u)
  z = _tc_layer4(s_p, u, dis, W4, b4, w_head)

  out = _tc_pool(z, batch.reshape(_N, 1), c0)
  return out
